# baseline (speedup 1.0000x reference)
import functools
import numpy as np
import jax
import jax.numpy as jnp
from jax import lax
from jax.experimental import pallas as pl
from jax.experimental.pallas import tpu as pltpu
from jax.experimental.pallas import tpu_sc as plsc

_NDET = 100
_Y0 = 100
_NB = 8
_NC = 84
_LANES = 128
_W0 = 96
_SLAB = _NC * _LANES
_OUTW = 112

_SEL_BATCH = np.array(
    [0, 0, 0, 0, 0, 0, 0, 0, 0, 0, 0, 0, 0, 1, 1, 1, 1, 1, 1, 1, 1, 1,
     1, 1, 1, 1, 1, 1, 2, 2, 2, 2, 2, 2, 2, 2, 2, 2, 2, 3, 3, 3, 3, 3,
     3, 3, 3, 3, 3, 3, 3, 3, 3, 3, 3, 4, 4, 4, 4, 4, 4, 4, 4, 4, 4, 4,
     4, 4, 4, 4, 4, 5, 5, 5, 5, 5, 5, 5, 5, 5, 5, 6, 6, 6, 6, 6, 6, 6,
     6, 7, 7, 7, 7, 7, 7, 7, 7, 7, 7, 7], dtype=np.int32)

_N_GROUPS = 7
_GROUP_BASE = tuple(min(16 * w, 96) for w in range(_N_GROUPS))

_GROUP_RUNS = []
for _w in range(_N_GROUPS):
    _lo = _GROUP_BASE[_w]
    _dets = np.minimum(_lo + np.arange(16), _NDET - 1)
    _bs = _SEL_BATCH[_dets]
    _runs = [(int(_bs[0]), 0)]
    for _l in range(1, 16):
        if _bs[_l] != _runs[-1][0]:
            _runs.append((int(_bs[_l]), _l))
    _GROUP_RUNS.append(tuple(_runs))
_GROUP_RUNS = tuple(_GROUP_RUNS)

_MESH = plsc.VectorSubcoreMesh(
    core_axis_name="c", subcore_axis_name="s", num_cores=2, num_subcores=16)


@functools.partial(
    pl.kernel,
    out_type=jax.ShapeDtypeStruct((7 * _OUTW,), jnp.float32),
    mesh=_MESH,
    scratch_types=[
        pltpu.VMEM((3 * _SLAB,), jnp.float32),
        pltpu.VMEM((7 * 16,), jnp.float32),
    ],
)
def _sc_det(xs_hbm, out_hbm, buf, obuf):
    wid = lax.axis_index("s") * 2 + lax.axis_index("c")

    for w in range(_N_GROUPS):
        @pl.when(wid == w)
        def _(w=w):
            base = _GROUP_BASE[w]
            runs = _GROUP_RUNS[w]

            for j, (b, _) in enumerate(runs):
                pltpu.sync_copy(xs_hbm.at[pl.ds(b * _SLAB, _SLAB)],
                                buf.at[pl.ds(j * _SLAB, _SLAB)])

            lane = lax.broadcasted_iota(jnp.int32, (16,), 0)
            col0 = base + (_Y0 - _W0)

            def chan(c):
                g = None
                for j, (_, l0) in enumerate(runs):
                    v = buf[pl.ds(j * _SLAB + c * _LANES + col0, 16)]
                    g = v if g is None else jnp.where(lane >= l0, v, g)
                return g

            cx, cy, ww, hh = chan(0), chan(1), chan(2), chan(3)
            mx = chan(4)
            am = jnp.zeros((16,), jnp.int32)
            for c in range(5, _NC):
                g = chan(c)
                cond = g > mx
                am = jnp.where(cond, c - 4, am)
                mx = jnp.maximum(mx, g)

            bv = jnp.full((16,), runs[0][0], jnp.int32)
            for j in range(1, len(runs)):
                db = runs[j][0] - runs[j - 1][0]
                bv = bv + jnp.where(lane >= runs[j][1], db, 0)

            vals = (
                bv.astype(jnp.float32),
                cx - 0.5 * ww,
                cy - 0.5 * hh,
                cx + 0.5 * ww,
                cy + 0.5 * hh,
                am.astype(jnp.float32),
                mx,
            )
            for k, v in enumerate(vals):
                obuf[pl.ds(16 * k, 16)] = v
            for k in range(7):
                pltpu.sync_copy(obuf.at[pl.ds(16 * k, 16)],
                                out_hbm.at[pl.ds(k * _OUTW + base, 16)])


def kernel(x):
    xs = jax.lax.slice(x, (0, 0, _W0), (_NB, _NC, _W0 + _LANES))
    out_t = _sc_det(xs.reshape(_NB * _SLAB))
    return out_t.reshape(7, _OUTW)[:, :_NDET].T


# speedup vs baseline: 5.0482x; 5.0482x over previous
import numpy as np
import jax
import jax.numpy as jnp
from jax import lax
from jax.experimental import pallas as pl
from jax.experimental.pallas import tpu as pltpu

_NDET = 100
_Y0 = 100
_NB = 8
_NC = 84
_LANES = 100
_W0 = 100

_SEL_BATCH = np.array(
    [0, 0, 0, 0, 0, 0, 0, 0, 0, 0, 0, 0, 0, 1, 1, 1, 1, 1, 1, 1, 1, 1,
     1, 1, 1, 1, 1, 1, 2, 2, 2, 2, 2, 2, 2, 2, 2, 2, 2, 3, 3, 3, 3, 3,
     3, 3, 3, 3, 3, 3, 3, 3, 3, 3, 3, 4, 4, 4, 4, 4, 4, 4, 4, 4, 4, 4,
     4, 4, 4, 4, 4, 5, 5, 5, 5, 5, 5, 5, 5, 5, 5, 6, 6, 6, 6, 6, 6, 6,
     6, 7, 7, 7, 7, 7, 7, 7, 7, 7, 7, 7], dtype=np.int32)

_RUN_STARTS = tuple(int(s) for s in np.flatnonzero(np.diff(_SEL_BATCH)) + 1)


def _det_kernel(x_ref, o_ref):
    data = x_ref[...][:, :, _Y0 - _W0:_Y0 - _W0 + _NDET]

    col = lax.broadcasted_iota(jnp.int32, (1, _NDET), 1)
    bsel = jnp.zeros((1, _NDET), jnp.int32)
    for s in _RUN_STARTS:
        bsel = bsel + (col >= s).astype(jnp.int32)

    sel = data[0]
    for b in range(1, _NB):
        sel = jnp.where(bsel == b, data[b], sel)

    cx, cy = sel[0:1], sel[1:2]
    w, h = sel[2:3], sel[3:4]
    x1 = cx - 0.5 * w
    y1 = cy - 0.5 * h
    x2 = cx + 0.5 * w
    y2 = cy + 0.5 * h

    scores = sel[4:_NC]
    mx = jnp.max(scores, axis=0, keepdims=True)
    ids = lax.broadcasted_iota(jnp.int32, scores.shape, 0)
    am = jnp.min(jnp.where(scores == mx, ids, _NC), axis=0, keepdims=True)

    xf = bsel.astype(jnp.float32)
    out7 = jnp.concatenate(
        [xf, x1, y1, x2, y2, am.astype(jnp.float32), mx], axis=0)
    o_ref[...] = out7.T


def kernel(x):
    xs = jax.lax.slice(x, (0, 0, _W0), (_NB, _NC, _W0 + _LANES))
    return pl.pallas_call(
        _det_kernel,
        out_shape=jax.ShapeDtypeStruct((_NDET, 7), jnp.float32),
    )(xs)


# speedup vs baseline: 5.0588x; 1.0021x over previous
import numpy as np
import jax
import jax.numpy as jnp
from jax import lax
from jax.experimental import pallas as pl

_NDET = 100
_Y0 = 100
_NB = 8
_NC = 84
_LANES = 100
_W0 = 100

_SEL_BATCH = np.array(
    [0, 0, 0, 0, 0, 0, 0, 0, 0, 0, 0, 0, 0, 1, 1, 1, 1, 1, 1, 1, 1, 1,
     1, 1, 1, 1, 1, 1, 2, 2, 2, 2, 2, 2, 2, 2, 2, 2, 2, 3, 3, 3, 3, 3,
     3, 3, 3, 3, 3, 3, 3, 3, 3, 3, 3, 4, 4, 4, 4, 4, 4, 4, 4, 4, 4, 4,
     4, 4, 4, 4, 4, 5, 5, 5, 5, 5, 5, 5, 5, 5, 5, 6, 6, 6, 6, 6, 6, 6,
     6, 7, 7, 7, 7, 7, 7, 7, 7, 7, 7, 7], dtype=np.int32)

_RUN_STARTS = tuple(int(s) for s in np.flatnonzero(np.diff(_SEL_BATCH)) + 1)


def _det_kernel(x_ref, o_ref):
    data = x_ref[...][:, :, _Y0 - _W0:_Y0 - _W0 + _NDET]

    col = lax.broadcasted_iota(jnp.int32, (1, _NDET), 1)
    bsel = jnp.zeros((1, _NDET), jnp.int32)
    for s in _RUN_STARTS:
        bsel = bsel + (col >= s).astype(jnp.int32)

    sel = data[0]
    for b in range(1, _NB):
        sel = jnp.where(bsel == b, data[b], sel)

    cx, cy = sel[0:1], sel[1:2]
    w, h = sel[2:3], sel[3:4]
    x1 = cx - 0.5 * w
    y1 = cy - 0.5 * h
    x2 = cx + 0.5 * w
    y2 = cy + 0.5 * h

    scores = sel[4:_NC]
    mx = jnp.max(scores, axis=0, keepdims=True)
    ids = lax.broadcasted_iota(jnp.int32, scores.shape, 0)
    am = jnp.min(jnp.where(scores == mx, ids, _NC), axis=0, keepdims=True)

    xf = bsel.astype(jnp.float32)
    out7 = jnp.concatenate(
        [xf, x1, y1, x2, y2, am.astype(jnp.float32), mx], axis=0)
    o_ref[...] = out7.T


def kernel(x):
    xs = jax.lax.slice(x, (0, 0, _W0), (_NB, _NC, _W0 + _LANES))
    return pl.pallas_call(
        _det_kernel,
        out_shape=jax.ShapeDtypeStruct((_NDET, 7), jnp.float32),
    )(xs)
